# baseline (speedup 1.0000x reference)
import functools

import jax
import jax.numpy as jnp
import numpy as np
from jax import lax
from jax.experimental import pallas as pl
from jax.experimental.pallas import tpu as pltpu
from jax.experimental.pallas import tpu_sc as plsc

NUM_N = 10000
HIDDEN = 128
PF = 0.2
PE = 0.2
NOISE_STD = 0.1
E_PER_MP = 160000
E_TOT = 2 * E_PER_MP + NUM_N

NC = 2
NS = 16
NW = NC * NS
L = 16

C = 128
GC = 6
NG = 14
NCH = NG * GC
NBUF = 2
GROWS = 24
P = NCH * C
E_PAD = NW * P

HISTP = 10240
HNS = HISTP // NS

def _rand_consts():
    rk = jax.random.key(123)
    kf, ke, kn = jax.random.split(rk, 3)
    col_keep = (
        ~(jax.random.uniform(kf, (HIDDEN,)) < PF)
    ).astype(jnp.float32).reshape(1, HIDDEN)
    emask = jnp.concatenate([
        (jax.random.uniform(ke, (2 * E_PER_MP,)) > PE).astype(jnp.float32),
        jnp.ones((NUM_N,), jnp.float32),
        jnp.zeros((E_PAD - E_TOT,), jnp.float32),
    ]).reshape(NW, NCH, C)
    noise = jax.random.normal(kn, (NUM_N, HIDDEN), dtype=jnp.float32) * NOISE_STD
    return col_keep, emask, noise

_mesh = plsc.VectorSubcoreMesh(core_axis_name="c", subcore_axis_name="s")


@functools.partial(
    pl.kernel,
    out_type=(
        jax.ShapeDtypeStruct((NC, HISTP), jnp.float32),
        jax.ShapeDtypeStruct((NW, NCH, C), jnp.float32),
    ),
    mesh=_mesh,
    scratch_types=[
        pltpu.VMEM((NCH, C), jnp.int32),
        pltpu.VMEM((NCH, C), jnp.float32),
        pltpu.VMEM((NCH, C), jnp.float32),
        pltpu.VMEM((HISTP,), jnp.float32),
        pltpu.VMEM((NS, HNS), jnp.float32),
        pltpu.VMEM_SHARED((NS, HISTP), jnp.float32),
    ],
    compiler_params=pltpu.CompilerParams(needs_layout_passes=False),
)
def _deg_kernel(dstg, wraw, emaskg, degp, wmask, dst_v, w_v, m_v, hist, red,
                shared):
    cid = lax.axis_index("c")
    sid = lax.axis_index("s")
    wid = sid * NC + cid
    pltpu.sync_copy(dstg.at[wid], dst_v)
    pltpu.sync_copy(wraw.at[wid], w_v)
    pltpu.sync_copy(emaskg.at[wid], m_v)

    def mask_body(i, _):
        for f in range(C // L):
            sl = pl.ds(f * L, L)
            w_v[i, sl] = w_v[i, sl] * m_v[i, sl]
        return 0

    lax.fori_loop(0, NCH, mask_body, 0)
    pltpu.sync_copy(w_v, wmask.at[wid])

    def zero_body(i, _):
        hist[pl.ds(i * L, L)] = jnp.zeros((L,), jnp.float32)
        return 0

    lax.fori_loop(0, HISTP // L, zero_body, 0)

    def ch_body(ch, _):
        def g_body(g, _):
            sl = pl.ds(g * L, L)
            plsc.addupdate_scatter(hist, [dst_v[ch, sl]], w_v[ch, sl])
            return 0

        lax.fori_loop(0, C // L, g_body, 0)
        return 0

    lax.fori_loop(0, NCH, ch_body, 0)

    pltpu.sync_copy(hist, shared.at[sid])
    plsc.subcore_barrier()
    pltpu.sync_copy(shared.at[:, pl.ds(sid * HNS, HNS)], red)

    def red_body(f, _):
        sl = pl.ds(f * L, L)
        acc = red[0, sl]
        for k in range(1, NS):
            acc = acc + red[k, sl]
        red[0, sl] = acc
        return 0

    lax.fori_loop(0, HNS // L, red_body, 0)
    pltpu.sync_copy(red.at[0], degp.at[cid, pl.ds(sid * HNS, HNS)])


@functools.partial(
    pl.kernel,
    out_type=jax.ShapeDtypeStruct((NC, NUM_N, HIDDEN), jnp.float32),
    mesh=_mesh,
    scratch_types=[
        pltpu.VMEM((2 * GROWS, C), jnp.int32),
        pltpu.VMEM((NBUF, C, HIDDEN), jnp.float32),
        pltpu.SemaphoreType.DMA((NBUF,)),
        pltpu.SemaphoreType.DMA((2,)),
        pltpu.VMEM_SHARED((NUM_N, HIDDEN), jnp.float32),
    ],
    compiler_params=pltpu.CompilerParams(needs_layout_passes=False),
)
def _layer_kernel(h2, sdw, out, ibuf, rbuf, gsem, isem, acc):
    cid = lax.axis_index("c")
    sid = lax.axis_index("s")
    wid = sid * NC + cid

    def zero_body(i, _):
        for f in range(HIDDEN // L):
            rbuf[0, i, pl.ds(f * L, L)] = jnp.zeros((L,), jnp.float32)
        return 0

    lax.fori_loop(0, C, zero_body, 0)
    NB = NUM_N // 40
    for j in range((NB + NS - 1) // NS):
        blk = j * NS + sid

        @pl.when(blk < NB)
        def _():
            pltpu.sync_copy(rbuf.at[0, pl.ds(0, 40)], acc.at[pl.ds(blk * 40, 40)])

    plsc.subcore_barrier()

    def _wait_rows(sem_arr, b):
        pltpu.make_async_copy(h2.at[pl.ds(0, C)], rbuf.at[b], sem_arr.at[b]).wait()

    def _launch_grp(g, slot):
        pltpu.async_copy(sdw.at[wid, g],
                         ibuf.at[pl.ds(slot * GROWS, GROWS)], isem.at[slot])

    def _wait_grp(slot):
        pltpu.make_async_copy(sdw.at[wid, 0],
                              ibuf.at[pl.ds(slot * GROWS, GROWS)],
                              isem.at[slot]).wait()

    def _launch_gather(slot, cc, b):
        pltpu.async_copy(h2.at[ibuf.at[slot * GROWS + cc * 3]], rbuf.at[b],
                         gsem.at[b])

    _launch_grp(0, 0)
    _wait_grp(0)
    _launch_gather(0, 0, 0)

    def _one_group(g, gs):
        ns = 1 - gs
        _launch_grp(g + 1, ns)
        for cc in range(GC):
            b = cc % NBUF
            if cc < GC - 1:
                _launch_gather(gs, cc + 1, (b + 1) % NBUF)
            else:
                _wait_grp(ns)
                _launch_gather(ns, 0, (b + 1) % NBUF)
            _wait_rows(gsem, b)

            def g_body(f8, _):
                wv = plsc.bitcast(
                    ibuf[gs * GROWS + cc * 3 + 2, pl.ds(f8 * L, L)],
                    jnp.float32)
                for e in range(L):
                    nb = jnp.full((L,), wv[e], jnp.float32)
                    row = f8 * L + e
                    for f in range(HIDDEN // L):
                        sl = pl.ds(f * L, L)
                        rbuf[b, row, sl] = rbuf[b, row, sl] * nb
                return 0

            lax.fori_loop(0, C // L, g_body, 0)
            pltpu.sync_copy(rbuf.at[b],
                            acc.at[ibuf.at[gs * GROWS + cc * 3 + 1]], add=True)

    @pl.loop(0, NG, step=2)
    def _(g0):
        _one_group(g0, 0)
        _one_group(g0 + 1, 1)

    _wait_rows(gsem, 0)
    plsc.subcore_barrier()
    for j in range((NB + NS - 1) // NS):
        blk = j * NS + sid

        @pl.when(blk < NB)
        def _():
            sl = pl.ds(blk * 40, 40)
            pltpu.sync_copy(acc.at[sl], out.at[cid, sl])


_BR = 2000


def _prep_mm_body(feat, noise, colmask, dinv, W, o_ref):
    x = feat[...] * colmask[...] + noise[...]
    h = jnp.dot(x, W[...], preferred_element_type=jnp.float32)
    o_ref[...] = h * dinv[...]


def _mid_body(aggA, aggB, dinv, b, W, o_ref):
    x = jnp.maximum((aggA[0] + aggB[0]) * dinv[...] + b[...], 0.0)
    h = jnp.dot(x, W[...], preferred_element_type=jnp.float32)
    o_ref[...] = h * dinv[...]


def _final_body(aggA, aggB, dinv, b, o_ref):
    o_ref[...] = jnp.maximum((aggA[0] + aggB[0]) * dinv[...] + b[...], 0.0)


_bspec = pl.BlockSpec((_BR, HIDDEN), lambda i: (i, 0))
_dspec = pl.BlockSpec((_BR, 1), lambda i: (i, 0))
_aspecA = pl.BlockSpec((1, _BR, HIDDEN), lambda i: (0, i, 0))
_aspecB = pl.BlockSpec((1, _BR, HIDDEN), lambda i: (1, i, 0))
_vspec = pl.BlockSpec((1, HIDDEN), lambda i: (0, 0))
_wspec = pl.BlockSpec((HIDDEN, HIDDEN), lambda i: (0, 0))
_GRID = NUM_N // _BR
_OSHAPE = jax.ShapeDtypeStruct((NUM_N, HIDDEN), jnp.float32)


def _prep_mm(feat, noise, col_keep, dinv, W):
    return pl.pallas_call(
        _prep_mm_body,
        grid=(_GRID,),
        in_specs=[_bspec, _bspec, _vspec, _dspec, _wspec],
        out_specs=_bspec,
        out_shape=_OSHAPE,
    )(feat, noise, col_keep, dinv, W)


def _mid(aggp, dinv, b, W):
    return pl.pallas_call(
        _mid_body,
        grid=(_GRID,),
        in_specs=[_aspecA, _aspecB, _dspec, _vspec, _wspec],
        out_specs=_bspec,
        out_shape=_OSHAPE,
    )(aggp, aggp, dinv, b.reshape(1, HIDDEN), W)


def _final(aggp, dinv, b):
    return pl.pallas_call(
        _final_body,
        grid=(_GRID,),
        in_specs=[_aspecA, _aspecB, _dspec, _vspec],
        out_specs=_bspec,
        out_shape=_OSHAPE,
    )(aggp, aggp, dinv, b.reshape(1, HIDDEN))


def kernel(feat, mp0_indices, mp0_values, mp1_indices, mp1_values, W0, b0, W1, b1):
    si = jnp.arange(NUM_N, dtype=jnp.int32)
    zpad = jnp.zeros((E_PAD - E_TOT,), jnp.int32)
    src = jnp.concatenate(
        [mp0_indices[0].astype(jnp.int32), mp1_indices[0].astype(jnp.int32),
         si, zpad]
    ).reshape(NW, NCH, C)
    dst = jnp.concatenate(
        [mp0_indices[1].astype(jnp.int32), mp1_indices[1].astype(jnp.int32),
         si, zpad]
    ).reshape(NW, NCH, C)
    wraw = jnp.concatenate(
        [mp0_values, mp1_values, jnp.ones((NUM_N,), jnp.float32),
         jnp.zeros((E_PAD - E_TOT,), jnp.float32)]
    ).reshape(NW, NCH, C)

    col_keep, emask, noise = _rand_consts()
    degp, wg = _deg_kernel(dst, wraw, emask)
    deg = degp[0, :NUM_N] + degp[1, :NUM_N]
    dinv = jnp.where(deg > 0, lax.rsqrt(deg), 0.0).reshape(NUM_N, 1)

    wbits = lax.bitcast_convert_type(wg, jnp.int32)
    sdw = jnp.stack([src, dst, wbits], axis=2).reshape(NW, NG, GC * 3, C)
    sdw = jnp.pad(sdw, ((0, 0), (0, 1), (0, GROWS - GC * 3), (0, 0)))

    h2 = _prep_mm(feat, noise, col_keep, dinv, W0)
    aggp = _layer_kernel(h2, sdw)
    h2b = _mid(aggp, dinv, b0, W1)
    aggp2 = _layer_kernel(h2b, sdw)
    return _final(aggp2, dinv, b1)


# speedup vs baseline: 2.4176x; 2.4176x over previous
import functools

import jax
import jax.numpy as jnp
import numpy as np
from jax import lax
from jax.experimental import pallas as pl
from jax.experimental.pallas import tpu as pltpu
from jax.experimental.pallas import tpu_sc as plsc

NUM_N = 10000
HIDDEN = 128
PF = 0.2
PE = 0.2
NOISE_STD = 0.1
E_PER_MP = 160000
E_TOT = 2 * E_PER_MP + NUM_N

NC = 2
NS = 16
NW = NC * NS
L = 16

C = 128
NCH = 81
P = NCH * C
E_PAD = NW * P

HISTP = 10240
HNS = HISTP // NS

def _rand_consts():
    rk = jax.random.key(123)
    kf, ke, kn = jax.random.split(rk, 3)
    col_keep = (
        ~(jax.random.uniform(kf, (HIDDEN,)) < PF)
    ).astype(jnp.float32).reshape(1, HIDDEN)
    emask = jnp.concatenate([
        (jax.random.uniform(ke, (2 * E_PER_MP,)) > PE).astype(jnp.float32),
        jnp.ones((NUM_N,), jnp.float32),
        jnp.zeros((E_PAD - E_TOT,), jnp.float32),
    ]).reshape(NW, NCH, C)
    noise = jax.random.normal(kn, (NUM_N, HIDDEN), dtype=jnp.float32) * NOISE_STD
    return col_keep, emask, noise

_mesh = plsc.VectorSubcoreMesh(core_axis_name="c", subcore_axis_name="s")


@functools.partial(
    pl.kernel,
    out_type=(
        jax.ShapeDtypeStruct((NC, HISTP), jnp.float32),
        jax.ShapeDtypeStruct((NW, NCH, C), jnp.float32),
    ),
    mesh=_mesh,
    scratch_types=[
        pltpu.VMEM((NCH, C), jnp.int32),
        pltpu.VMEM((NCH, C), jnp.float32),
        pltpu.VMEM((NCH, C), jnp.float32),
        pltpu.VMEM((HISTP,), jnp.float32),
        pltpu.VMEM((NS, HNS), jnp.float32),
        pltpu.VMEM_SHARED((NS, HISTP), jnp.float32),
    ],
    compiler_params=pltpu.CompilerParams(needs_layout_passes=False),
)
def _deg_kernel(dstg, wraw, emaskg, degp, wmask, dst_v, w_v, m_v, hist, red,
                shared):
    cid = lax.axis_index("c")
    sid = lax.axis_index("s")
    wid = sid * NC + cid
    pltpu.sync_copy(dstg.at[wid], dst_v)
    pltpu.sync_copy(wraw.at[wid], w_v)
    pltpu.sync_copy(emaskg.at[wid], m_v)

    def mask_body(i, _):
        for f in range(C // L):
            sl = pl.ds(f * L, L)
            w_v[i, sl] = w_v[i, sl] * m_v[i, sl]
        return 0

    lax.fori_loop(0, NCH, mask_body, 0)
    pltpu.sync_copy(w_v, wmask.at[wid])

    def zero_body(i, _):
        hist[pl.ds(i * L, L)] = jnp.zeros((L,), jnp.float32)
        return 0

    lax.fori_loop(0, HISTP // L, zero_body, 0)

    def ch_body(ch, _):
        def g_body(g, _):
            sl = pl.ds(g * L, L)
            plsc.addupdate_scatter(hist, [dst_v[ch, sl]], w_v[ch, sl])
            return 0

        lax.fori_loop(0, C // L, g_body, 0)
        return 0

    lax.fori_loop(0, NCH, ch_body, 0)

    pltpu.sync_copy(hist, shared.at[sid])
    plsc.subcore_barrier()
    pltpu.sync_copy(shared.at[:, pl.ds(sid * HNS, HNS)], red)

    def red_body(f, _):
        sl = pl.ds(f * L, L)
        acc = red[0, sl]
        for k in range(1, NS):
            acc = acc + red[k, sl]
        red[0, sl] = acc
        return 0

    lax.fori_loop(0, HNS // L, red_body, 0)
    pltpu.sync_copy(red.at[0], degp.at[cid, pl.ds(sid * HNS, HNS)])


@functools.partial(
    pl.kernel,
    out_type=jax.ShapeDtypeStruct((NC, NUM_N, HIDDEN), jnp.float32),
    mesh=_mesh,
    scratch_types=[
        pltpu.VMEM((NCH, C), jnp.int32),
        pltpu.VMEM((NCH, C), jnp.int32),
        pltpu.VMEM((NCH, C), jnp.float32),
        pltpu.VMEM((C, HIDDEN), jnp.float32),
        pltpu.SemaphoreType.DMA,
        pltpu.VMEM_SHARED((NUM_N, HIDDEN), jnp.float32),
    ],
    compiler_params=pltpu.CompilerParams(needs_layout_passes=False),
)
def _layer_kernel(h2, srcg, dstg, wg, out, src_v, dst_v, w_v, rbuf, gsem, acc):
    cid = lax.axis_index("c")
    sid = lax.axis_index("s")
    wid = sid * NC + cid
    pltpu.sync_copy(srcg.at[wid], src_v)
    pltpu.sync_copy(dstg.at[wid], dst_v)
    pltpu.sync_copy(wg.at[wid], w_v)

    def zero_body(i, _):
        for f in range(HIDDEN // L):
            rbuf[i, pl.ds(f * L, L)] = jnp.zeros((L,), jnp.float32)
        return 0

    lax.fori_loop(0, C, zero_body, 0)
    NB = NUM_N // 80
    for j in range((NB + NS - 1) // NS):
        blk = j * NS + sid

        @pl.when(blk < NB)
        def _():
            pltpu.sync_copy(rbuf.at[pl.ds(0, 80)], acc.at[pl.ds(blk * 80, 80)])

    plsc.subcore_barrier()

    def ch_body(ch, _):
        pltpu.async_copy(h2.at[src_v.at[ch]], rbuf, gsem).wait()

        def g_body(g, _):
            wv = w_v[ch, pl.ds(g * L, L)]
            for e in range(L):
                nb = jnp.full((L,), wv[e], jnp.float32)
                row = g * L + e
                for f in range(HIDDEN // L):
                    sl = pl.ds(f * L, L)
                    rbuf[row, sl] = rbuf[row, sl] * nb
            return 0

        lax.fori_loop(0, C // L, g_body, 0)
        pltpu.sync_copy(rbuf, acc.at[dst_v.at[ch]], add=True)
        return 0

    lax.fori_loop(0, NCH, ch_body, 0)
    plsc.subcore_barrier()
    for j in range((NB + NS - 1) // NS):
        blk = j * NS + sid

        @pl.when(blk < NB)
        def _():
            sl = pl.ds(blk * 80, 80)
            pltpu.sync_copy(acc.at[sl], out.at[cid, sl])


_BR = 2000


def _prep_mm_body(feat, noise, colmask, dinv, W, o_ref):
    x = feat[...] * colmask[...] + noise[...]
    h = jnp.dot(x, W[...], preferred_element_type=jnp.float32)
    o_ref[...] = h * dinv[...]


def _mid_body(aggA, aggB, dinv, b, W, o_ref):
    x = jnp.maximum((aggA[0] + aggB[0]) * dinv[...] + b[...], 0.0)
    h = jnp.dot(x, W[...], preferred_element_type=jnp.float32)
    o_ref[...] = h * dinv[...]


def _final_body(aggA, aggB, dinv, b, o_ref):
    o_ref[...] = jnp.maximum((aggA[0] + aggB[0]) * dinv[...] + b[...], 0.0)


_bspec = pl.BlockSpec((_BR, HIDDEN), lambda i: (i, 0))
_dspec = pl.BlockSpec((_BR, 1), lambda i: (i, 0))
_aspecA = pl.BlockSpec((1, _BR, HIDDEN), lambda i: (0, i, 0))
_aspecB = pl.BlockSpec((1, _BR, HIDDEN), lambda i: (1, i, 0))
_vspec = pl.BlockSpec((1, HIDDEN), lambda i: (0, 0))
_wspec = pl.BlockSpec((HIDDEN, HIDDEN), lambda i: (0, 0))
_GRID = NUM_N // _BR
_OSHAPE = jax.ShapeDtypeStruct((NUM_N, HIDDEN), jnp.float32)


def _prep_mm(feat, noise, col_keep, dinv, W):
    return pl.pallas_call(
        _prep_mm_body,
        grid=(_GRID,),
        in_specs=[_bspec, _bspec, _vspec, _dspec, _wspec],
        out_specs=_bspec,
        out_shape=_OSHAPE,
    )(feat, noise, col_keep, dinv, W)


def _mid(aggp, dinv, b, W):
    return pl.pallas_call(
        _mid_body,
        grid=(_GRID,),
        in_specs=[_aspecA, _aspecB, _dspec, _vspec, _wspec],
        out_specs=_bspec,
        out_shape=_OSHAPE,
    )(aggp, aggp, dinv, b.reshape(1, HIDDEN), W)


def _final(aggp, dinv, b):
    return pl.pallas_call(
        _final_body,
        grid=(_GRID,),
        in_specs=[_aspecA, _aspecB, _dspec, _vspec],
        out_specs=_bspec,
        out_shape=_OSHAPE,
    )(aggp, aggp, dinv, b.reshape(1, HIDDEN))


def kernel(feat, mp0_indices, mp0_values, mp1_indices, mp1_values, W0, b0, W1, b1):
    si = jnp.arange(NUM_N, dtype=jnp.int32)
    zpad = jnp.zeros((E_PAD - E_TOT,), jnp.int32)
    src = jnp.concatenate(
        [mp0_indices[0].astype(jnp.int32), mp1_indices[0].astype(jnp.int32),
         si, zpad]
    ).reshape(NW, NCH, C)
    dst = jnp.concatenate(
        [mp0_indices[1].astype(jnp.int32), mp1_indices[1].astype(jnp.int32),
         si, zpad]
    ).reshape(NW, NCH, C)
    wraw = jnp.concatenate(
        [mp0_values, mp1_values, jnp.ones((NUM_N,), jnp.float32),
         jnp.zeros((E_PAD - E_TOT,), jnp.float32)]
    ).reshape(NW, NCH, C)

    col_keep, emask, noise = _rand_consts()
    degp, wg = _deg_kernel(dst, wraw, emask)
    deg = degp[0, :NUM_N] + degp[1, :NUM_N]
    dinv = jnp.where(deg > 0, lax.rsqrt(deg), 0.0).reshape(NUM_N, 1)

    h2 = _prep_mm(feat, noise, col_keep, dinv, W0)
    aggp = _layer_kernel(h2, src, dst, wg)
    h2b = _mid(aggp, dinv, b0, W1)
    aggp2 = _layer_kernel(h2b, src, dst, wg)
    return _final(aggp2, dinv, b1)
